# baseline (speedup 1.0000x reference)
import functools

import jax
import jax.numpy as jnp
from jax import lax
from jax.experimental import pallas as pl
from jax.experimental.pallas import tpu as pltpu
from jax.experimental.pallas import tpu_sc as plsc

NN = 10000
EE = 320000
DH = 128

NW = 32
RPW = 384
NPAD = NW * RPW
DUMMY_ROW = 383

EPW = EE // NW
EBUF = 10112
BCAP = 512
BSTR = BCAP + 16
ACAP = 10240

ECHUNK = 512
GATHER = 128


def _bin_body(src_hbm, dst_hbm, asrc_hbm, arow_hbm, cnt_hbm,
              sbuf, dbuf, bsrc, brow, cvec, cur, acur):
    cid = lax.axis_index("c")
    sid = lax.axis_index("s")
    wid = sid * 2 + cid
    iota = lax.iota(jnp.int32, 16)

    def zcur(b, _):
        cur[b] = 0
        acur[b] = 0
        return 0
    lax.fori_loop(0, NW, zcur, 0)

    base = wid * EPW
    ab = pl.multiple_of((base // 128) * 128, 128)
    off0 = base - ab
    pltpu.sync_copy(src_hbm.at[pl.ds(ab, EBUF)], sbuf.at[pl.ds(0, EBUF)])
    pltpu.sync_copy(dst_hbm.at[pl.ds(ab, EBUF)], dbuf.at[pl.ds(0, EBUF)])

    def edge(j, _):
        d = dbuf[pl.ds(j, 16)][0]
        s = sbuf[pl.ds(j, 16)][0]
        b = d & 31
        r = lax.shift_right_logical(d, 5)
        cb = cur[b]
        pos = b * BSTR + cb
        bsrc[pl.ds(pos, 16)] = jnp.full((16,), s, jnp.int32)
        brow[pl.ds(pos, 16)] = jnp.full((16,), r, jnp.int32)
        cb1 = cb + 1

        @pl.when(cb1 >= BCAP)
        def _():
            a = acur[b]
            abase = pl.multiple_of((wid * NW + b) * ACAP + a, 128)
            pltpu.sync_copy(bsrc.at[pl.ds(b * BSTR, BCAP)],
                            asrc_hbm.at[pl.ds(abase, BCAP)])
            pltpu.sync_copy(brow.at[pl.ds(b * BSTR, BCAP)],
                            arow_hbm.at[pl.ds(abase, BCAP)])
            acur[b] = a + BCAP

        cur[b] = jnp.where(cb1 >= BCAP, 0, cb1)
        return 0

    lax.fori_loop(off0, off0 + EPW, edge, 0)

    def drain(b, _):
        cb = cur[b]
        pos = b * BSTR + cb
        bsrc[pl.ds(pos, 16)] = jnp.zeros((16,), jnp.int32)
        brow[pl.ds(pos, 16)] = jnp.full((16,), DUMMY_ROW, jnp.int32)
        a = acur[b]

        @pl.when(cb > 0)
        def _():
            abase = pl.multiple_of((wid * NW + b) * ACAP + a, 128)
            pltpu.sync_copy(bsrc.at[pl.ds(b * BSTR, BCAP)],
                            asrc_hbm.at[pl.ds(abase, BCAP)])
            pltpu.sync_copy(brow.at[pl.ds(b * BSTR, BCAP)],
                            arow_hbm.at[pl.ds(abase, BCAP)])

        cvec[pl.ds(b, 16)] = jnp.full((16,), a + cb, jnp.int32)
        return 0

    lax.fori_loop(0, NW, drain, 0)
    pltpu.sync_copy(cvec.at[pl.ds(0, 128)], cnt_hbm.at[pl.ds(pl.multiple_of(wid * 128, 128), 128)])


_MESH = plsc.VectorSubcoreMesh(core_axis_name="c", subcore_axis_name="s")


def _bin_edges(src, dst):
    return pl.kernel(
        _bin_body,
        out_type=(
            jax.ShapeDtypeStruct((NW * NW * ACAP,), jnp.int32),
            jax.ShapeDtypeStruct((NW * NW * ACAP,), jnp.int32),
            jax.ShapeDtypeStruct((NW * 128,), jnp.int32),
        ),
        mesh=_MESH,
        scratch_types=[
            pltpu.VMEM((EBUF + 16,), jnp.int32),
            pltpu.VMEM((EBUF + 16,), jnp.int32),
            pltpu.VMEM((NW * BSTR,), jnp.int32),
            pltpu.VMEM((NW * BSTR,), jnp.int32),
            pltpu.VMEM((128 + 16,), jnp.int32),
            pltpu.SMEM((NW,), jnp.int32),
            pltpu.SMEM((NW,), jnp.int32),
        ],
    )(src, dst)


def _acc_body(hp_hbm, asrc_hbm, arow_hbm, cnt_hbm, out_hbm,
              csrc, crow, msga, msgb, acc, cntv, sem, semb):
    cid = lax.axis_index("c")
    sid = lax.axis_index("s")
    wid = sid * 2 + cid
    iota = lax.iota(jnp.int32, 16)

    def zero_acc(i, _):
        for f in range(DH // 16):
            acc[i, pl.ds(f * 16, 16)] = jnp.zeros((16,), jnp.float32)
        return 0
    lax.fori_loop(0, RPW, zero_acc, 0)

    pltpu.sync_copy(cnt_hbm, cntv.at[pl.ds(0, NW * 128)])

    def per_u(u, _):
        cnt = cntv[pl.ds(u * 128 + wid, 16)][0]
        nch = (cnt + ECHUNK - 1) // ECHUNK

        def per_chunk(k, _):
            cbase = k * ECHUNK
            abase = pl.multiple_of((u * NW + wid) * ACAP + cbase, 128)
            pltpu.sync_copy(asrc_hbm.at[pl.ds(abase, ECHUNK)],
                            csrc.at[pl.ds(0, ECHUNK)])
            pltpu.sync_copy(arow_hbm.at[pl.ds(abase, ECHUNK)],
                            crow.at[pl.ds(0, ECHUNK)])
            rem = cnt - cbase
            for t in range(ECHUNK // 16):
                lane = t * 16 + iota
                valid = lane < rem
                sl = pl.ds(t * 16, 16)
                s0 = csrc[sl]
                s0 = jnp.minimum(jnp.maximum(s0, 0), NN - 1)
                csrc[sl] = s0
                r0 = crow[sl]
                r0 = jnp.where(valid, r0,
                               jnp.full((16,), DUMMY_ROW, jnp.int32))
                r0 = jnp.minimum(jnp.maximum(r0, 0), DUMMY_ROW)
                crow[sl] = r0

            ng = (jnp.minimum(rem, ECHUNK) + GATHER - 1) // GATHER

            def fire(t, buf, sm):
                idxs = csrc.at[pl.ds(t * GATHER, GATHER)]
                pltpu.async_copy(hp_hbm.at[idxs], buf, sm)

            def drain(buf, sm):
                idxs = csrc.at[pl.ds(0, GATHER)]
                pltpu.make_async_copy(hp_hbm.at[idxs], buf, sm).wait()

            def accum_chunk(t, buf):
                tb = t * GATHER

                def body(jo, _):
                    j = tb + jo
                    r = crow[pl.ds(j, 16)][0]
                    for f in range(DH // 16):
                        sl = pl.ds(f * 16, 16)
                        acc[r, sl] = jnp.maximum(acc[r, sl], buf[jo, sl])
                    return 0
                lax.fori_loop(0, GATHER, body, 0, unroll=4)

            @pl.when(ng > 0)
            def _():
                fire(0, msga, sem)

            for p in range(ECHUNK // GATHER // 2):
                t0 = 2 * p
                t1 = t0 + 1

                @pl.when(t0 < ng)
                def _(t0=t0, t1=t1):
                    @pl.when(t1 < ng)
                    def _():
                        fire(t1, msgb, semb)
                    drain(msga, sem)
                    accum_chunk(t0, msga)

                @pl.when(t1 < ng)
                def _(t0=t0, t1=t1):
                    @pl.when(t1 + 1 < ng)
                    def _():
                        fire(min(t1 + 1, ECHUNK // GATHER - 1), msga, sem)
                    drain(msgb, semb)
                    accum_chunk(t1, msgb)
            return 0

        lax.fori_loop(0, nch, per_chunk, 0)
        return 0

    lax.fori_loop(0, NW, per_u, 0)

    obase = pl.multiple_of(wid * RPW, 128)
    pltpu.sync_copy(acc.at[pl.ds(0, RPW)],
                    out_hbm.at[pl.ds(obase, RPW)])


def _segmax(hp, asrc, arow, cnt):
    out = pl.kernel(
        _acc_body,
        out_type=jax.ShapeDtypeStruct((NW * RPW, DH), jnp.float32),
        mesh=_MESH,
        scratch_types=[
            pltpu.VMEM((ECHUNK + 16,), jnp.int32),
            pltpu.VMEM((ECHUNK + 16,), jnp.int32),
            pltpu.VMEM((GATHER, DH), jnp.float32),
            pltpu.VMEM((GATHER, DH), jnp.float32),
            pltpu.VMEM((RPW, DH), jnp.float32),
            pltpu.VMEM((NW * 128 + 16,), jnp.int32),
            pltpu.SemaphoreType.DMA,
            pltpu.SemaphoreType.DMA,
        ],
    )(hp, asrc, arow, cnt)
    seg = out.reshape(NW, RPW, DH).transpose(1, 0, 2).reshape(NPAD, DH)
    return seg


def _mm_relu(x, W, b):
    M, K = x.shape
    Nd = W.shape[1]
    BM = 2000

    def body(x_ref, w_ref, b_ref, o_ref):
        o_ref[...] = jax.nn.relu(
            jnp.dot(x_ref[...], w_ref[...],
                    preferred_element_type=jnp.float32) + b_ref[...])

    return pl.pallas_call(
        body,
        grid=(M // BM,),
        in_specs=[
            pl.BlockSpec((BM, K), lambda i: (i, 0)),
            pl.BlockSpec((K, Nd), lambda i: (0, 0)),
            pl.BlockSpec((1, Nd), lambda i: (0, 0)),
        ],
        out_specs=pl.BlockSpec((BM, Nd), lambda i: (i, 0)),
        out_shape=jax.ShapeDtypeStruct((M, Nd), jnp.float32),
    )(x, W, b.reshape(1, Nd))


def _sage_out(x, seg, hp, Ws, Wn, b):
    M, K = x.shape
    Nd = Ws.shape[1]
    BM = 2000

    def body(x_ref, seg_ref, hp_ref, ws_ref, wn_ref, b_ref, o_ref):
        hn = jnp.maximum(seg_ref[...], hp_ref[...])
        o_ref[...] = jax.nn.relu(
            jnp.dot(x_ref[...], ws_ref[...],
                    preferred_element_type=jnp.float32)
            + jnp.dot(hn, wn_ref[...], preferred_element_type=jnp.float32)
            + b_ref[...])

    return pl.pallas_call(
        body,
        grid=(M // BM,),
        in_specs=[
            pl.BlockSpec((BM, K), lambda i: (i, 0)),
            pl.BlockSpec((BM, K), lambda i: (i, 0)),
            pl.BlockSpec((BM, K), lambda i: (i, 0)),
            pl.BlockSpec((K, Nd), lambda i: (0, 0)),
            pl.BlockSpec((K, Nd), lambda i: (0, 0)),
            pl.BlockSpec((1, Nd), lambda i: (0, 0)),
        ],
        out_specs=pl.BlockSpec((BM, Nd), lambda i: (i, 0)),
        out_shape=jax.ShapeDtypeStruct((M, Nd), jnp.float32),
    )(x, seg, hp, Ws, Wn, b.reshape(1, Nd))


def kernel(features, edge_index, Wp1, bp1, Ws1, Wn1, b1,
           Wp2, bp2, Ws2, Wn2, b2):
    pad = jnp.zeros((2, 128), jnp.int32)
    epad = jnp.concatenate([edge_index, pad], axis=1)
    src = epad[0]
    dst = epad[1]
    asrc, arow, cnt = _bin_edges(src, dst)
    hp1 = _mm_relu(features, Wp1, bp1)
    seg1 = _segmax(hp1, asrc, arow, cnt)[:NN]
    h1 = _sage_out(features, seg1, hp1, Ws1, Wn1, b1)
    hp2 = _mm_relu(h1, Wp2, bp2)
    seg2 = _segmax(hp2, asrc, arow, cnt)[:NN]
    return _sage_out(h1, seg2, hp2, Ws2, Wn2, b2)


# speedup vs baseline: 1.1637x; 1.1637x over previous
import functools

import jax
import jax.numpy as jnp
from jax import lax
from jax.experimental import pallas as pl
from jax.experimental.pallas import tpu as pltpu
from jax.experimental.pallas import tpu_sc as plsc

NN = 10000
EE = 320000
DH = 128

NW = 32
RPW = 384
NPAD = NW * RPW
DUMMY_ROW = 383

EPW = EE // NW
EBUF = 10112
BCAP = 512
BSTR = BCAP + 16
ACAP = 10240

ECHUNK = 512
GATHER = 128


def _bin_body(src_hbm, dst_hbm, asrc_hbm, arow_hbm, cnt_hbm,
              sbuf, dbuf, bsrc, brow, cvec, cur, acur):
    cid = lax.axis_index("c")
    sid = lax.axis_index("s")
    wid = sid * 2 + cid
    iota = lax.iota(jnp.int32, 16)

    def zcur(b, _):
        cur[b] = 0
        acur[b] = 0
        return 0
    lax.fori_loop(0, NW, zcur, 0)

    base = wid * EPW
    ab = pl.multiple_of((base // 128) * 128, 128)
    off0 = base - ab
    pltpu.sync_copy(src_hbm.at[pl.ds(ab, EBUF)], sbuf.at[pl.ds(0, EBUF)])
    pltpu.sync_copy(dst_hbm.at[pl.ds(ab, EBUF)], dbuf.at[pl.ds(0, EBUF)])

    def edge(j, _):
        d = dbuf[pl.ds(j, 16)][0]
        s = sbuf[pl.ds(j, 16)][0]
        b = d & 31
        r = lax.shift_right_logical(d, 5)
        cb = cur[b]
        pos = b * BSTR + cb
        bsrc[pl.ds(pos, 16)] = jnp.full((16,), s, jnp.int32)
        brow[pl.ds(pos, 16)] = jnp.full((16,), r, jnp.int32)
        cb1 = cb + 1

        @pl.when(cb1 >= BCAP)
        def _():
            a = acur[b]
            abase = pl.multiple_of((wid * NW + b) * ACAP + a, 128)
            pltpu.sync_copy(bsrc.at[pl.ds(b * BSTR, BCAP)],
                            asrc_hbm.at[pl.ds(abase, BCAP)])
            pltpu.sync_copy(brow.at[pl.ds(b * BSTR, BCAP)],
                            arow_hbm.at[pl.ds(abase, BCAP)])
            acur[b] = a + BCAP

        cur[b] = jnp.where(cb1 >= BCAP, 0, cb1)
        return 0

    lax.fori_loop(off0, off0 + EPW, edge, 0)

    def drain(b, _):
        cb = cur[b]
        pos = b * BSTR + cb
        bsrc[pl.ds(pos, 16)] = jnp.zeros((16,), jnp.int32)
        brow[pl.ds(pos, 16)] = jnp.full((16,), DUMMY_ROW, jnp.int32)
        a = acur[b]

        @pl.when(cb > 0)
        def _():
            abase = pl.multiple_of((wid * NW + b) * ACAP + a, 128)
            pltpu.sync_copy(bsrc.at[pl.ds(b * BSTR, BCAP)],
                            asrc_hbm.at[pl.ds(abase, BCAP)])
            pltpu.sync_copy(brow.at[pl.ds(b * BSTR, BCAP)],
                            arow_hbm.at[pl.ds(abase, BCAP)])

        cvec[pl.ds(b, 16)] = jnp.full((16,), a + cb, jnp.int32)
        return 0

    lax.fori_loop(0, NW, drain, 0)
    pltpu.sync_copy(cvec.at[pl.ds(0, 128)], cnt_hbm.at[pl.ds(pl.multiple_of(wid * 128, 128), 128)])


_MESH = plsc.VectorSubcoreMesh(core_axis_name="c", subcore_axis_name="s")


def _bin_edges(src, dst):
    return pl.kernel(
        _bin_body,
        out_type=(
            jax.ShapeDtypeStruct((NW * NW * ACAP,), jnp.int32),
            jax.ShapeDtypeStruct((NW * NW * ACAP,), jnp.int32),
            jax.ShapeDtypeStruct((NW * 128,), jnp.int32),
        ),
        mesh=_MESH,
        scratch_types=[
            pltpu.VMEM((EBUF + 16,), jnp.int32),
            pltpu.VMEM((EBUF + 16,), jnp.int32),
            pltpu.VMEM((NW * BSTR,), jnp.int32),
            pltpu.VMEM((NW * BSTR,), jnp.int32),
            pltpu.VMEM((128 + 16,), jnp.int32),
            pltpu.SMEM((NW,), jnp.int32),
            pltpu.SMEM((NW,), jnp.int32),
        ],
    )(src, dst)


def _acc_body(hp_hbm, asrc_hbm, arow_hbm, cnt_hbm, out_hbm,
              csrc, crow, msga, msgb, acc, cntv, oidx, sem, semb):
    cid = lax.axis_index("c")
    sid = lax.axis_index("s")
    wid = sid * 2 + cid
    iota = lax.iota(jnp.int32, 16)

    def zero_acc(i, _):
        for f in range(DH // 16):
            acc[i, pl.ds(f * 16, 16)] = jnp.zeros((16,), jnp.float32)
        return 0
    lax.fori_loop(0, RPW, zero_acc, 0)

    pltpu.sync_copy(cnt_hbm, cntv.at[pl.ds(0, NW * 128)])

    def per_u(u, _):
        cnt = cntv[pl.ds(u * 128 + wid, 16)][0]
        nch = (cnt + ECHUNK - 1) // ECHUNK

        def per_chunk(k, _):
            cbase = k * ECHUNK
            abase = pl.multiple_of((u * NW + wid) * ACAP + cbase, 128)
            pltpu.sync_copy(asrc_hbm.at[pl.ds(abase, ECHUNK)],
                            csrc.at[pl.ds(0, ECHUNK)])
            pltpu.sync_copy(arow_hbm.at[pl.ds(abase, ECHUNK)],
                            crow.at[pl.ds(0, ECHUNK)])
            rem = cnt - cbase
            for t in range(ECHUNK // 16):
                lane = t * 16 + iota
                valid = lane < rem
                sl = pl.ds(t * 16, 16)
                s0 = csrc[sl]
                s0 = jnp.minimum(jnp.maximum(s0, 0), NN - 1)
                csrc[sl] = s0
                r0 = crow[sl]
                r0 = jnp.where(valid, r0,
                               jnp.full((16,), DUMMY_ROW, jnp.int32))
                r0 = jnp.minimum(jnp.maximum(r0, 0), DUMMY_ROW)
                crow[sl] = r0

            ng = (jnp.minimum(rem, ECHUNK) + GATHER - 1) // GATHER

            def fire(t, buf, sm):
                idxs = csrc.at[pl.ds(t * GATHER, GATHER)]
                pltpu.async_copy(hp_hbm.at[idxs], buf, sm)

            def drain(buf, sm):
                idxs = csrc.at[pl.ds(0, GATHER)]
                pltpu.make_async_copy(hp_hbm.at[idxs], buf, sm).wait()

            def accum_chunk(t, buf):
                tb = t * GATHER

                def body(jo, _):
                    j = tb + jo
                    r = crow[pl.ds(j, 16)][0]
                    for f in range(DH // 16):
                        sl = pl.ds(f * 16, 16)
                        acc[r, sl] = jnp.maximum(acc[r, sl], buf[jo, sl])
                    return 0
                lax.fori_loop(0, GATHER, body, 0, unroll=4)

            @pl.when(ng > 0)
            def _():
                fire(0, msga, sem)

            for p in range(ECHUNK // GATHER // 2):
                t0 = 2 * p
                t1 = t0 + 1

                @pl.when(t0 < ng)
                def _(t0=t0, t1=t1):
                    @pl.when(t1 < ng)
                    def _():
                        fire(t1, msgb, semb)
                    drain(msga, sem)
                    accum_chunk(t0, msga)

                @pl.when(t1 < ng)
                def _(t0=t0, t1=t1):
                    @pl.when(t1 + 1 < ng)
                    def _():
                        fire(min(t1 + 1, ECHUNK // GATHER - 1), msga, sem)
                    drain(msgb, semb)
                    accum_chunk(t1, msgb)
            return 0

        lax.fori_loop(0, nch, per_chunk, 0)
        return 0

    lax.fori_loop(0, NW, per_u, 0)

    for t in range(RPW // 128):
        for t2 in range(8):
            oidx[t, pl.ds(t2 * 16, 16)] = wid + 32 * (t * 128 + t2 * 16
                                                      + iota)
    for t in range(RPW // 128):
        pltpu.async_copy(acc.at[pl.ds(t * 128, 128)],
                         out_hbm.at[oidx.at[t]], sem).wait()


def _segmax(hp, asrc, arow, cnt):
    out = pl.kernel(
        _acc_body,
        out_type=jax.ShapeDtypeStruct((NPAD, DH), jnp.float32),
        mesh=_MESH,
        scratch_types=[
            pltpu.VMEM((ECHUNK + 16,), jnp.int32),
            pltpu.VMEM((ECHUNK + 16,), jnp.int32),
            pltpu.VMEM((GATHER, DH), jnp.float32),
            pltpu.VMEM((GATHER, DH), jnp.float32),
            pltpu.VMEM((RPW, DH), jnp.float32),
            pltpu.VMEM((NW * 128 + 16,), jnp.int32),
            pltpu.VMEM((RPW // 128, 128), jnp.int32),
            pltpu.SemaphoreType.DMA,
            pltpu.SemaphoreType.DMA,
        ],
    )(hp, asrc, arow, cnt)
    return out


def _mm_relu(x, W, b):
    M, K = x.shape
    Nd = W.shape[1]
    BM = 2000

    def body(x_ref, w_ref, b_ref, o_ref):
        o_ref[...] = jax.nn.relu(
            jnp.dot(x_ref[...], w_ref[...],
                    preferred_element_type=jnp.float32) + b_ref[...])

    return pl.pallas_call(
        body,
        grid=(M // BM,),
        in_specs=[
            pl.BlockSpec((BM, K), lambda i: (i, 0)),
            pl.BlockSpec((K, Nd), lambda i: (0, 0)),
            pl.BlockSpec((1, Nd), lambda i: (0, 0)),
        ],
        out_specs=pl.BlockSpec((BM, Nd), lambda i: (i, 0)),
        out_shape=jax.ShapeDtypeStruct((M, Nd), jnp.float32),
    )(x, W, b.reshape(1, Nd))


def _sage_out(x, seg, hp, Ws, Wn, b):
    M, K = x.shape
    Nd = Ws.shape[1]
    BM = 2000

    def body(x_ref, seg_ref, hp_ref, ws_ref, wn_ref, b_ref, o_ref):
        hn = jnp.maximum(seg_ref[...], hp_ref[...])
        o_ref[...] = jax.nn.relu(
            jnp.dot(x_ref[...], ws_ref[...],
                    preferred_element_type=jnp.float32)
            + jnp.dot(hn, wn_ref[...], preferred_element_type=jnp.float32)
            + b_ref[...])

    return pl.pallas_call(
        body,
        grid=(M // BM,),
        in_specs=[
            pl.BlockSpec((BM, K), lambda i: (i, 0)),
            pl.BlockSpec((BM, K), lambda i: (i, 0)),
            pl.BlockSpec((BM, K), lambda i: (i, 0)),
            pl.BlockSpec((K, Nd), lambda i: (0, 0)),
            pl.BlockSpec((K, Nd), lambda i: (0, 0)),
            pl.BlockSpec((1, Nd), lambda i: (0, 0)),
        ],
        out_specs=pl.BlockSpec((BM, Nd), lambda i: (i, 0)),
        out_shape=jax.ShapeDtypeStruct((M, Nd), jnp.float32),
    )(x, seg, hp, Ws, Wn, b.reshape(1, Nd))


def kernel(features, edge_index, Wp1, bp1, Ws1, Wn1, b1,
           Wp2, bp2, Ws2, Wn2, b2):
    pad = jnp.zeros((2, 128), jnp.int32)
    epad = jnp.concatenate([edge_index, pad], axis=1)
    src = epad[0]
    dst = epad[1]
    asrc, arow, cnt = _bin_edges(src, dst)
    hp1 = _mm_relu(features, Wp1, bp1)
    seg1 = _segmax(hp1, asrc, arow, cnt)[:NN]
    h1 = _sage_out(features, seg1, hp1, Ws1, Wn1, b1)
    hp2 = _mm_relu(h1, Wp2, bp2)
    seg2 = _segmax(hp2, asrc, arow, cnt)[:NN]
    return _sage_out(h1, seg2, hp2, Ws2, Wn2, b2)
